# baseline (speedup 1.0000x reference)
import functools

import jax
import jax.numpy as jnp
import numpy as np
from jax.experimental import pallas as pl
from jax.experimental.pallas import tpu as pltpu
from jax.sharding import Mesh, PartitionSpec as P

_B = 128
_V = 100000
_BC = 2048

_KS0 = np.uint32(0)
_KS1 = np.uint32(42)
_KS2 = np.uint32(np.uint32(0) ^ np.uint32(42) ^ np.uint32(0x1BD11BDA))

_TINY = np.float32(np.finfo(np.float32).tiny)
_NEG_INF = np.float32(-np.inf)
_BIG_IDX = np.int32(0x7FFFFFFF)


def _rotl(x, d):
    return jax.lax.shift_left(x, np.uint32(d)) | jax.lax.shift_right_logical(
        x, np.uint32(32 - d)
    )


def _threefry_bits(x1):
    rot_a = (13, 15, 26, 6)
    rot_b = (17, 29, 16, 24)

    def rounds(x0, x1, rots):
        for r in rots:
            x0 = x0 + x1
            x1 = _rotl(x1, r)
            x1 = x1 ^ x0
        return x0, x1

    x0 = x1
    x1 = _rotl(x1, 13) ^ x0
    x0, x1 = rounds(x0, x1, (15, 26, 6))
    x0 = x0 + _KS1
    x1 = x1 + np.uint32(_KS2 + np.uint32(1))
    x0, x1 = rounds(x0, x1, rot_b)
    x0 = x0 + _KS2
    x1 = x1 + np.uint32(_KS0 + np.uint32(2))
    x0, x1 = rounds(x0, x1, rot_a)
    x0 = x0 + _KS0
    x1 = x1 + np.uint32(_KS1 + np.uint32(3))
    x0, x1 = rounds(x0, x1, rot_b)
    x0 = x0 + _KS1
    x1 = x1 + np.uint32(_KS2 + np.uint32(4))
    x0, x1 = rounds(x0, x1, rot_a)
    x0 = x0 + _KS2
    x1 = x1 + np.uint32(_KS0 + np.uint32(5))
    return x0 ^ x1


def _body(vloc, nblk, hbm_ref, tail_ref, off_ref, oi_ref, ov_ref, buf_ref,
          flat_ref, rv_ref, ri_ref, sem_ref):
    lane = jax.lax.broadcasted_iota(jnp.int32, (_B, _BC), 1)
    row = jax.lax.broadcasted_iota(jnp.int32, (_B, _BC), 0)
    flat_ref[...] = row * _V + lane + off_ref[0] + jnp.int32(_KS1)
    rv_ref[...] = jnp.full((_B, _BC), _NEG_INF, jnp.float32)
    ri_ref[...] = jnp.zeros((_B, _BC), jnp.int32)

    def base_of(j):
        return jnp.minimum(j * _BC, vloc - _BC)

    def start_copy(j, slot):
        @pl.when(j < nblk - 1)
        def _():
            pltpu.make_async_copy(
                hbm_ref.at[:, pl.ds(j * _BC, _BC)],
                buf_ref.at[slot],
                sem_ref.at[slot],
            ).start()

        @pl.when(j == nblk - 1)
        def _():
            pltpu.make_async_copy(
                tail_ref, buf_ref.at[slot], sem_ref.at[slot]
            ).start()

    def wait_copy(slot):
        pltpu.make_async_copy(
            hbm_ref.at[:, pl.ds(0, _BC)],
            buf_ref.at[slot],
            sem_ref.at[slot],
        ).wait()

    start_copy(0, 0)

    def step(j, carry):
        slot = jax.lax.rem(j, 2)

        @pl.when(j + 1 < nblk)
        def _():
            start_copy(j + 1, 1 - slot)

        wait_copy(slot)
        x = buf_ref[slot]
        f42 = (flat_ref[...] + base_of(j)).astype(jnp.uint32)

        bits = _threefry_bits(f42)
        fbits = jax.lax.shift_right_logical(bits, np.uint32(9)) | np.uint32(0x3F800000)
        flt = jax.lax.bitcast_convert_type(fbits, jnp.float32) - np.float32(1.0)
        u = jnp.maximum(flt, _TINY)
        g = -jnp.log(-jnp.log(u))
        val = g + x

        better = val > rv_ref[...]
        rv_ref[...] = jnp.where(better, val, rv_ref[...])
        ri_ref[...] = jnp.where(better, f42.astype(jnp.int32), ri_ref[...])
        return 0

    jax.lax.fori_loop(0, nblk, step, 0)

    rv = rv_ref[...]
    col = (
        ri_ref[...]
        - jnp.int32(_KS1)
        - jax.lax.broadcasted_iota(jnp.int32, (_B, _BC), 0) * _V
    )
    bm = jnp.max(rv, axis=1, keepdims=True)
    oi_ref[...] = jnp.min(jnp.where(rv == bm, col, _BIG_IDX), axis=1, keepdims=True)
    ov_ref[...] = bm


@functools.lru_cache(maxsize=None)
def _make_local(vloc):
    nblk = (vloc + _BC - 1) // _BC

    call = pl.pallas_call(
        functools.partial(_body, vloc, nblk),
        in_specs=[
            pl.BlockSpec(memory_space=pl.ANY),
            pl.BlockSpec(memory_space=pl.ANY),
            pl.BlockSpec(memory_space=pltpu.SMEM),
        ],
        out_specs=(
            pl.BlockSpec((_B, 1), memory_space=pltpu.VMEM),
            pl.BlockSpec((_B, 1), memory_space=pltpu.VMEM),
        ),
        out_shape=(
            jax.ShapeDtypeStruct((_B, 1), jnp.int32),
            jax.ShapeDtypeStruct((_B, 1), jnp.float32),
        ),
        scratch_shapes=[
            pltpu.VMEM((2, _B, _BC), jnp.float32),
            pltpu.VMEM((_B, _BC), jnp.int32),
            pltpu.VMEM((_B, _BC), jnp.float32),
            pltpu.VMEM((_B, _BC), jnp.int32),
            pltpu.SemaphoreType.DMA((2,)),
        ],
    )

    def local(logits_loc, off):
        tail = jax.lax.slice(logits_loc, (0, vloc - _BC), (_B, vloc))
        idx, val = call(logits_loc, tail, jnp.reshape(off, (1,)).astype(jnp.int32))
        return idx.reshape(_B), val.reshape(_B)

    return local


def kernel(logits):
    devs = jax.devices()
    nshard = 2 if (len(devs) >= 2 and _V % 2 == 0) else 1
    if nshard == 1:
        idx, _ = _make_local(_V)(logits, jnp.int32(0))
        return idx

    vloc = _V // nshard
    local = _make_local(vloc)
    mesh = Mesh(np.array(devs[:nshard]), ("x",))

    def shard_fn(logits_loc):
        off = jax.lax.axis_index("x").astype(jnp.int32) * vloc
        idx, val = local(logits_loc, off)
        vals = jax.lax.all_gather(val, "x")
        idxs = jax.lax.all_gather(idx, "x")
        best_v = vals[0]
        best_i = idxs[0]
        for s in range(1, nshard):
            take = vals[s] > best_v
            best_v = jnp.where(take, vals[s], best_v)
            best_i = jnp.where(take, idxs[s], best_i)
        return best_i

    return jax.shard_map(
        shard_fn,
        mesh=mesh,
        in_specs=P(None, "x"),
        out_specs=P(),
        check_vma=False,
    )(logits)


# speedup vs baseline: 2.1746x; 2.1746x over previous
import jax
import jax.numpy as jnp
import numpy as np
from jax.experimental import pallas as pl
from jax.experimental.pallas import tpu as pltpu

_B = 128
_V = 100000
_BC = 2048
_NBLK = (_V + _BC - 1) // _BC

_KS0 = np.uint32(0)
_KS1 = np.uint32(42)
_KS2 = np.uint32(np.uint32(0) ^ np.uint32(42) ^ np.uint32(0x1BD11BDA))

_TINY = np.float32(np.finfo(np.float32).tiny)
_NEG_INF = np.float32(-np.inf)
_BIG_IDX = np.int32(0x7FFFFFFF)


def _rotl(x, d):
    return jax.lax.shift_left(x, np.uint32(d)) | jax.lax.shift_right_logical(
        x, np.uint32(32 - d)
    )


def _threefry_bits(x1):
    rot_a = (13, 15, 26, 6)
    rot_b = (17, 29, 16, 24)

    def rounds(x0, x1, rots):
        for r in rots:
            x0 = x0 + x1
            x1 = _rotl(x1, r)
            x1 = x1 ^ x0
        return x0, x1

    x0 = x1
    x1 = _rotl(x1, 13) ^ x0
    x0, x1 = rounds(x0, x1, (15, 26, 6))
    x0 = x0 + _KS1
    x1 = x1 + np.uint32(_KS2 + np.uint32(1))
    x0, x1 = rounds(x0, x1, rot_b)
    x0 = x0 + _KS2
    x1 = x1 + np.uint32(_KS0 + np.uint32(2))
    x0, x1 = rounds(x0, x1, rot_a)
    x0 = x0 + _KS0
    x1 = x1 + np.uint32(_KS1 + np.uint32(3))
    x0, x1 = rounds(x0, x1, rot_b)
    x0 = x0 + _KS1
    x1 = x1 + np.uint32(_KS2 + np.uint32(4))
    x0, x1 = rounds(x0, x1, rot_a)
    x0 = x0 + _KS2
    x1 = x1 + np.uint32(_KS0 + np.uint32(5))
    return x0 ^ x1


def _body(hbm_ref, tail_ref, out_ref, buf_ref, flat_ref, rv_ref, ri_ref, sem_ref):
    lane = jax.lax.broadcasted_iota(jnp.int32, (_B, _BC), 1)
    row = jax.lax.broadcasted_iota(jnp.int32, (_B, _BC), 0)
    flat_ref[...] = row * _V + lane + jnp.int32(_KS1)
    rv_ref[...] = jnp.full((_B, 1), _NEG_INF, jnp.float32)
    ri_ref[...] = jnp.zeros((_B, 1), jnp.int32)

    def start_copy(j, slot):
        @pl.when(j < _NBLK - 1)
        def _():
            pltpu.make_async_copy(
                hbm_ref.at[:, pl.ds(j * _BC, _BC)],
                buf_ref.at[slot],
                sem_ref.at[slot],
            ).start()

        @pl.when(j == _NBLK - 1)
        def _():
            pltpu.make_async_copy(
                tail_ref, buf_ref.at[slot], sem_ref.at[slot]
            ).start()

    def wait_copy(slot):
        pltpu.make_async_copy(
            hbm_ref.at[:, pl.ds(0, _BC)],
            buf_ref.at[slot],
            sem_ref.at[slot],
        ).wait()

    start_copy(0, 0)

    def step(j, carry):
        slot = jax.lax.rem(j, 2)

        @pl.when(j + 1 < _NBLK)
        def _():
            start_copy(j + 1, 1 - slot)

        wait_copy(slot)
        x = buf_ref[slot]
        f42 = (flat_ref[...] + jnp.minimum(j * _BC, _V - _BC)).astype(jnp.uint32)

        bits = _threefry_bits(f42)
        fbits = jax.lax.shift_right_logical(bits, np.uint32(9)) | np.uint32(0x3F800000)
        flt = jax.lax.bitcast_convert_type(fbits, jnp.float32) - np.float32(1.0)
        u = jnp.maximum(flt, _TINY)
        g = -jnp.log(-jnp.log(u))
        val = g + x

        bm = jnp.max(val, axis=1, keepdims=True)
        bi = jnp.min(
            jnp.where(val == bm, f42.astype(jnp.int32), _BIG_IDX),
            axis=1,
            keepdims=True,
        )
        better = bm > rv_ref[...]
        rv_ref[...] = jnp.where(better, bm, rv_ref[...])
        ri_ref[...] = jnp.where(better, bi, ri_ref[...])
        return 0

    jax.lax.fori_loop(0, _NBLK, step, 0)

    out_ref[...] = (
        ri_ref[...]
        - jnp.int32(_KS1)
        - jax.lax.broadcasted_iota(jnp.int32, (_B, 1), 0) * _V
    )


def kernel(logits):
    tail = jax.lax.slice(logits, (0, _V - _BC), (_B, _V))
    out = pl.pallas_call(
        _body,
        in_specs=[
            pl.BlockSpec(memory_space=pl.ANY),
            pl.BlockSpec(memory_space=pl.ANY),
        ],
        out_specs=pl.BlockSpec((_B, 1), memory_space=pltpu.VMEM),
        out_shape=jax.ShapeDtypeStruct((_B, 1), jnp.int32),
        scratch_shapes=[
            pltpu.VMEM((2, _B, _BC), jnp.float32),
            pltpu.VMEM((_B, _BC), jnp.int32),
            pltpu.VMEM((_B, 1), jnp.float32),
            pltpu.VMEM((_B, 1), jnp.int32),
            pltpu.SemaphoreType.DMA((2,)),
        ],
    )(logits, tail)
    return out.reshape(_B)


# speedup vs baseline: 2.2343x; 1.0275x over previous
import jax
import jax.numpy as jnp
import numpy as np
from jax.experimental import pallas as pl
from jax.experimental.pallas import tpu as pltpu

_B = 128
_V = 100000
_BC = 2048
_BD = 4096
_ND = (_V + _BD - 1) // _BD

_KS0 = np.uint32(0)
_KS1 = np.uint32(42)
_KS2 = np.uint32(np.uint32(0) ^ np.uint32(42) ^ np.uint32(0x1BD11BDA))

_TINY = np.float32(np.finfo(np.float32).tiny)
_NEG_INF = np.float32(-np.inf)
_BIG_IDX = np.int32(0x7FFFFFFF)


def _rotl(x, d):
    return jax.lax.shift_left(x, np.uint32(d)) | jax.lax.shift_right_logical(
        x, np.uint32(32 - d)
    )


def _threefry_bits(x1):
    rot_a = (13, 15, 26, 6)
    rot_b = (17, 29, 16, 24)

    def rounds(x0, x1, rots):
        for r in rots:
            x0 = x0 + x1
            x1 = _rotl(x1, r)
            x1 = x1 ^ x0
        return x0, x1

    x0 = x1
    x1 = _rotl(x1, 13) ^ x0
    x0, x1 = rounds(x0, x1, (15, 26, 6))
    x0 = x0 + _KS1
    x1 = x1 + np.uint32(_KS2 + np.uint32(1))
    x0, x1 = rounds(x0, x1, rot_b)
    x0 = x0 + _KS2
    x1 = x1 + np.uint32(_KS0 + np.uint32(2))
    x0, x1 = rounds(x0, x1, rot_a)
    x0 = x0 + _KS0
    x1 = x1 + np.uint32(_KS1 + np.uint32(3))
    x0, x1 = rounds(x0, x1, rot_b)
    x0 = x0 + _KS1
    x1 = x1 + np.uint32(_KS2 + np.uint32(4))
    x0, x1 = rounds(x0, x1, rot_a)
    x0 = x0 + _KS2
    x1 = x1 + np.uint32(_KS0 + np.uint32(5))
    return x0 ^ x1


def _body(hbm_ref, tail_ref, out_ref, buf_ref, flat_ref, rv_ref, ri_ref, sem_ref):
    lane = jax.lax.broadcasted_iota(jnp.int32, (_B, _BC), 1)
    row = jax.lax.broadcasted_iota(jnp.int32, (_B, _BC), 0)
    flat_ref[...] = row * _V + lane + jnp.int32(_KS1)
    rv_ref[...] = jnp.full((_B, _BC), _NEG_INF, jnp.float32)
    ri_ref[...] = jnp.zeros((_B, _BC), jnp.int32)

    def start_copy(d, slot):
        @pl.when(d < _ND - 1)
        def _():
            pltpu.make_async_copy(
                hbm_ref.at[:, pl.ds(d * _BD, _BD)],
                buf_ref.at[slot],
                sem_ref.at[slot],
            ).start()

        @pl.when(d == _ND - 1)
        def _():
            pltpu.make_async_copy(
                tail_ref, buf_ref.at[slot], sem_ref.at[slot]
            ).start()

    def wait_copy(slot):
        pltpu.make_async_copy(
            hbm_ref.at[:, pl.ds(0, _BD)],
            buf_ref.at[slot],
            sem_ref.at[slot],
        ).wait()

    start_copy(0, 0)

    def step(d, carry):
        slot = jax.lax.rem(d, 2)

        @pl.when(d + 1 < _ND)
        def _():
            start_copy(d + 1, 1 - slot)

        wait_copy(slot)
        base0 = jnp.minimum(d * _BD, _V - _BD)
        for h in (0, 1):
            x = buf_ref[slot, :, h * _BC : (h + 1) * _BC]
            f42 = (flat_ref[...] + (base0 + h * _BC)).astype(jnp.uint32)

            bits = _threefry_bits(f42)
            fbits = jax.lax.shift_right_logical(bits, np.uint32(9)) | np.uint32(
                0x3F800000
            )
            flt = jax.lax.bitcast_convert_type(fbits, jnp.float32) - np.float32(1.0)
            u = jnp.maximum(flt, _TINY)
            val = x - jnp.log(-jnp.log(u))

            better = val > rv_ref[...]
            rv_ref[...] = jnp.where(better, val, rv_ref[...])
            ri_ref[...] = jnp.where(better, f42.astype(jnp.int32), ri_ref[...])
        return 0

    jax.lax.fori_loop(0, _ND, step, 0)

    rv = rv_ref[...]
    col = (
        ri_ref[...]
        - jnp.int32(_KS1)
        - jax.lax.broadcasted_iota(jnp.int32, (_B, _BC), 0) * _V
    )
    bm = jnp.max(rv, axis=1, keepdims=True)
    bi = jnp.min(jnp.where(rv == bm, col, _BIG_IDX), axis=1, keepdims=True)
    out_ref[...] = bi


def kernel(logits):
    tail = jax.lax.slice(logits, (0, _V - _BD), (_B, _V))
    out = pl.pallas_call(
        _body,
        in_specs=[
            pl.BlockSpec(memory_space=pl.ANY),
            pl.BlockSpec(memory_space=pl.ANY),
        ],
        out_specs=pl.BlockSpec((_B, 1), memory_space=pltpu.VMEM),
        out_shape=jax.ShapeDtypeStruct((_B, 1), jnp.int32),
        scratch_shapes=[
            pltpu.VMEM((2, _B, _BD), jnp.float32),
            pltpu.VMEM((_B, _BC), jnp.int32),
            pltpu.VMEM((_B, _BC), jnp.float32),
            pltpu.VMEM((_B, _BC), jnp.int32),
            pltpu.SemaphoreType.DMA((2,)),
        ],
    )(logits, tail)
    return out.reshape(_B)


# speedup vs baseline: 2.2674x; 1.0148x over previous
import jax
import jax.numpy as jnp
import numpy as np
from jax.experimental import pallas as pl
from jax.experimental.pallas import tpu as pltpu

_B = 128
_V = 100000
_BC = 2048
_NBLK = (_V + _BC - 1) // _BC

_KS0 = np.uint32(0)
_KS1 = np.uint32(42)
_KS2 = np.uint32(np.uint32(0) ^ np.uint32(42) ^ np.uint32(0x1BD11BDA))

_TINY = np.float32(np.finfo(np.float32).tiny)
_NEG_INF = np.float32(-np.inf)
_BIG_IDX = np.int32(0x7FFFFFFF)


def _rotl(x, d):
    return jax.lax.shift_left(x, np.uint32(d)) | jax.lax.shift_right_logical(
        x, np.uint32(32 - d)
    )


def _threefry_bits(x1):
    rot_a = (13, 15, 26, 6)
    rot_b = (17, 29, 16, 24)

    def rounds(x0, x1, rots):
        for r in rots:
            x0 = x0 + x1
            x1 = _rotl(x1, r)
            x1 = x1 ^ x0
        return x0, x1

    x0 = x1
    x1 = _rotl(x1, 13) ^ x0
    x0, x1 = rounds(x0, x1, (15, 26, 6))
    x0 = x0 + _KS1
    x1 = x1 + np.uint32(_KS2 + np.uint32(1))
    x0, x1 = rounds(x0, x1, rot_b)
    x0 = x0 + _KS2
    x1 = x1 + np.uint32(_KS0 + np.uint32(2))
    x0, x1 = rounds(x0, x1, rot_a)
    x0 = x0 + _KS0
    x1 = x1 + np.uint32(_KS1 + np.uint32(3))
    x0, x1 = rounds(x0, x1, rot_b)
    x0 = x0 + _KS1
    x1 = x1 + np.uint32(_KS2 + np.uint32(4))
    x0, x1 = rounds(x0, x1, rot_a)
    x0 = x0 + _KS2
    x1 = x1 + np.uint32(_KS0 + np.uint32(5))
    return x0 ^ x1


def _body(hbm_ref, tail_ref, out_ref, buf_ref, flat_ref, rv_ref, ri_ref, sem_ref):
    lane = jax.lax.broadcasted_iota(jnp.int32, (_B, _BC), 1)
    row = jax.lax.broadcasted_iota(jnp.int32, (_B, _BC), 0)
    flat_ref[...] = row * _V + lane + jnp.int32(_KS1)
    rv_ref[...] = jnp.full((_B, _BC), _NEG_INF, jnp.float32)
    ri_ref[...] = jnp.zeros((_B, _BC), jnp.int32)

    def base_of(j):
        return jnp.minimum(j * _BC, _V - _BC)

    def start_copy(j, slot):
        @pl.when(j < _NBLK - 1)
        def _():
            pltpu.make_async_copy(
                hbm_ref.at[:, pl.ds(j * _BC, _BC)],
                buf_ref.at[slot],
                sem_ref.at[slot],
            ).start()

        @pl.when(j == _NBLK - 1)
        def _():
            pltpu.make_async_copy(
                tail_ref, buf_ref.at[slot], sem_ref.at[slot]
            ).start()

    def wait_copy(slot):
        pltpu.make_async_copy(
            hbm_ref.at[:, pl.ds(0, _BC)],
            buf_ref.at[slot],
            sem_ref.at[slot],
        ).wait()

    start_copy(0, 0)

    def step(j, carry):
        slot = jax.lax.rem(j, 2)

        @pl.when(j + 1 < _NBLK)
        def _():
            start_copy(j + 1, 1 - slot)

        wait_copy(slot)
        x = buf_ref[slot]
        f42 = (flat_ref[...] + base_of(j)).astype(jnp.uint32)

        bits = _threefry_bits(f42)
        fbits = jax.lax.shift_right_logical(bits, np.uint32(9)) | np.uint32(0x3F800000)
        flt = jax.lax.bitcast_convert_type(fbits, jnp.float32) - np.float32(1.0)
        u = jnp.maximum(flt, _TINY)
        val = x - jnp.log(-jnp.log(u))

        better = val > rv_ref[...]
        rv_ref[...] = jnp.where(better, val, rv_ref[...])
        ri_ref[...] = jnp.where(better, f42.astype(jnp.int32), ri_ref[...])
        return 0

    jax.lax.fori_loop(0, _NBLK, step, 0, unroll=7)

    rv = rv_ref[...]
    col = (
        ri_ref[...]
        - jnp.int32(_KS1)
        - jax.lax.broadcasted_iota(jnp.int32, (_B, _BC), 0) * _V
    )
    bm = jnp.max(rv, axis=1, keepdims=True)
    bi = jnp.min(jnp.where(rv == bm, col, _BIG_IDX), axis=1, keepdims=True)
    out_ref[...] = bi


def kernel(logits):
    tail = jax.lax.slice(logits, (0, _V - _BC), (_B, _V))
    out = pl.pallas_call(
        _body,
        in_specs=[
            pl.BlockSpec(memory_space=pl.ANY),
            pl.BlockSpec(memory_space=pl.ANY),
        ],
        out_specs=pl.BlockSpec((_B, 1), memory_space=pltpu.VMEM),
        out_shape=jax.ShapeDtypeStruct((_B, 1), jnp.int32),
        scratch_shapes=[
            pltpu.VMEM((2, _B, _BC), jnp.float32),
            pltpu.VMEM((_B, _BC), jnp.int32),
            pltpu.VMEM((_B, _BC), jnp.float32),
            pltpu.VMEM((_B, _BC), jnp.int32),
            pltpu.SemaphoreType.DMA((2,)),
        ],
    )(logits, tail)
    return out.reshape(_B)


# speedup vs baseline: 2.3062x; 1.0171x over previous
import jax
import jax.numpy as jnp
import numpy as np
from jax.experimental import pallas as pl
from jax.experimental.pallas import tpu as pltpu

_B = 128
_V = 100000
_BC = 2048
_SUB = 128
_NBLK = (_V + _BC - 1) // _BC

_KS0 = np.uint32(0)
_KS1 = np.uint32(42)
_KS2 = np.uint32(np.uint32(0) ^ np.uint32(42) ^ np.uint32(0x1BD11BDA))

_TINY = np.float32(np.finfo(np.float32).tiny)
_NEG_INF = np.float32(-np.inf)
_BIG_IDX = np.int32(0x7FFFFFFF)


def _rotl(x, d):
    return jax.lax.shift_left(x, np.uint32(d)) | jax.lax.shift_right_logical(
        x, np.uint32(32 - d)
    )


def _threefry_bits(x1):
    rot_a = (13, 15, 26, 6)
    rot_b = (17, 29, 16, 24)

    def rounds(x0, x1, rots):
        for r in rots:
            x0 = x0 + x1
            x1 = _rotl(x1, r)
            x1 = x1 ^ x0
        return x0, x1

    x0 = x1
    x1 = _rotl(x1, 13) ^ x0
    x0, x1 = rounds(x0, x1, (15, 26, 6))
    x0 = x0 + _KS1
    x1 = x1 + np.uint32(_KS2 + np.uint32(1))
    x0, x1 = rounds(x0, x1, rot_b)
    x0 = x0 + _KS2
    x1 = x1 + np.uint32(_KS0 + np.uint32(2))
    x0, x1 = rounds(x0, x1, rot_a)
    x0 = x0 + _KS0
    x1 = x1 + np.uint32(_KS1 + np.uint32(3))
    x0, x1 = rounds(x0, x1, rot_b)
    x0 = x0 + _KS1
    x1 = x1 + np.uint32(_KS2 + np.uint32(4))
    x0, x1 = rounds(x0, x1, rot_a)
    x0 = x0 + _KS2
    x1 = x1 + np.uint32(_KS0 + np.uint32(5))
    return x0 ^ x1


def _body(hbm_ref, tail_ref, out_ref, buf_ref, flat_ref, rv_ref, ri_ref, sem_ref):
    lane = jax.lax.broadcasted_iota(jnp.int32, (_B, _BC), 1)
    row = jax.lax.broadcasted_iota(jnp.int32, (_B, _BC), 0)
    flat_ref[...] = row * _V + lane + jnp.int32(_KS1)
    rv_ref[...] = jnp.full((_B, _BC), _NEG_INF, jnp.float32)
    ri_ref[...] = jnp.zeros((_B, _BC), jnp.int32)

    def base_of(j):
        return jnp.minimum(j * _BC, _V - _BC)

    def start_copy(j, slot):
        @pl.when(j < _NBLK - 1)
        def _():
            pltpu.make_async_copy(
                hbm_ref.at[:, pl.ds(j * _BC, _BC)],
                buf_ref.at[slot],
                sem_ref.at[slot],
            ).start()

        @pl.when(j == _NBLK - 1)
        def _():
            pltpu.make_async_copy(
                tail_ref, buf_ref.at[slot], sem_ref.at[slot]
            ).start()

    def wait_copy(slot):
        pltpu.make_async_copy(
            hbm_ref.at[:, pl.ds(0, _BC)],
            buf_ref.at[slot],
            sem_ref.at[slot],
        ).wait()

    start_copy(0, 0)

    def step(j, carry):
        slot = jax.lax.rem(j, 2)

        @pl.when(j + 1 < _NBLK)
        def _():
            start_copy(j + 1, 1 - slot)

        wait_copy(slot)
        base = base_of(j)
        for s in range(_BC // _SUB):
            cs = slice(s * _SUB, (s + 1) * _SUB)
            x = buf_ref[slot, :, cs]
            f42 = (flat_ref[:, cs] + base).astype(jnp.uint32)

            bits = _threefry_bits(f42)
            fbits = jax.lax.shift_right_logical(bits, np.uint32(9)) | np.uint32(
                0x3F800000
            )
            flt = jax.lax.bitcast_convert_type(fbits, jnp.float32) - np.float32(1.0)
            u = jnp.maximum(flt, _TINY)
            val = x - jnp.log(-jnp.log(u))

            better = val > rv_ref[:, cs]
            rv_ref[:, cs] = jnp.where(better, val, rv_ref[:, cs])
            ri_ref[:, cs] = jnp.where(better, f42.astype(jnp.int32), ri_ref[:, cs])
        return 0

    jax.lax.fori_loop(0, _NBLK, step, 0, unroll=2)

    rv = rv_ref[...]
    col = (
        ri_ref[...]
        - jnp.int32(_KS1)
        - jax.lax.broadcasted_iota(jnp.int32, (_B, _BC), 0) * _V
    )
    bm = jnp.max(rv, axis=1, keepdims=True)
    bi = jnp.min(jnp.where(rv == bm, col, _BIG_IDX), axis=1, keepdims=True)
    out_ref[...] = bi


def kernel(logits):
    tail = jax.lax.slice(logits, (0, _V - _BC), (_B, _V))
    out = pl.pallas_call(
        _body,
        in_specs=[
            pl.BlockSpec(memory_space=pl.ANY),
            pl.BlockSpec(memory_space=pl.ANY),
        ],
        out_specs=pl.BlockSpec((_B, 1), memory_space=pltpu.VMEM),
        out_shape=jax.ShapeDtypeStruct((_B, 1), jnp.int32),
        scratch_shapes=[
            pltpu.VMEM((2, _B, _BC), jnp.float32),
            pltpu.VMEM((_B, _BC), jnp.int32),
            pltpu.VMEM((_B, _BC), jnp.float32),
            pltpu.VMEM((_B, _BC), jnp.int32),
            pltpu.SemaphoreType.DMA((2,)),
        ],
    )(logits, tail)
    return out.reshape(_B)
